# baseline (speedup 1.0000x reference)
import functools

import jax
import jax.numpy as jnp
from jax.experimental import pallas as pl
from jax.experimental.pallas import tpu as pltpu

B = 8
N = 1024
D_IN = 256
H = 128
G4 = 4 * H
CT = 256
NT = N // CT
UO = 16
NS = CT // UO

_f32 = jnp.float32
_bf16 = jnp.bfloat16


def _agg_gih0_kernel(adj_ref, xa_ref, x_ref, w_ref, b_ref, gf_ref, gb_ref):
    i = pl.program_id(0)
    w = w_ref[...]
    for b in range(B):
        mask = (adj_ref[b] > 0).astype(_bf16)
        ns = jnp.dot(mask, xa_ref[b], preferred_element_type=_f32)
        nsum = ns[:, :D_IN]
        deg = ns[:, D_IN:D_IN + 1]
        xb = x_ref[b, pl.ds(i * CT, CT), :]
        feat = (xb + nsum) / (1.0 + deg)
        g = jnp.dot(feat.astype(_bf16), w,
                    preferred_element_type=_f32) + b_ref[...]
        gf_ref[:, b, :] = g[:, :G4]
        gb_ref[:, b, :] = g[:, G4:]


def _sigmoid(x):
    return 0.5 + 0.5 * jnp.tanh(0.5 * x)


def _lstm_cell(g, c):
    i = _sigmoid(g[:, 0:H])
    f = _sigmoid(g[:, H:2 * H])
    gg = jnp.tanh(g[:, 2 * H:3 * H])
    o = _sigmoid(g[:, 3 * H:4 * H])
    c2 = f * c + i * gg
    h2 = o * jnp.tanh(c2)
    return h2, c2


def _scan_steps(gf_ref, gb_ref, wf, wb, carry0, store, extra=None):
    def dot2(h, w):
        hb = h.astype(_bf16)
        return (jnp.dot(hb, w[:, :2 * H], preferred_element_type=_f32),
                jnp.dot(hb, w[:, 2 * H:], preferred_element_type=_f32))

    def cell2(rg0, rg1, g, c):
        i = _sigmoid(rg0[:, 0:H] + g[:, 0:H])
        f = _sigmoid(rg0[:, H:2 * H] + g[:, H:2 * H])
        gg = jnp.tanh(rg1[:, 0:H] + g[:, 2 * H:3 * H])
        o = _sigmoid(rg1[:, H:2 * H] + g[:, 3 * H:4 * H])
        c2 = f * c + i * gg
        return o * jnp.tanh(c2), c2

    def step(s, carry):
        h_f, c_f, c_b, rgb0, rgb1 = carry
        rgf0, rgf1 = dot2(h_f, wf)
        h_b, c_b = cell2(rgb0, rgb1, gb_ref[CT - 1 - s], c_b)
        rgb0, rgb1 = dot2(h_b, wb)
        h_f, c_f = cell2(rgf0, rgf1, gf_ref[s], c_f)
        if store is not None:
            store(s, h_f, h_b)
        return (h_f, c_f, c_b, rgb0, rgb1)

    def outer(j, carry):
        if extra is not None:
            extra(j)
        for k in range(UO):
            carry = step(j * UO + k, carry)
        return carry

    return jax.lax.fori_loop(0, CT // UO, outer, carry0)


def _init_or_load(hf_scr, cf_scr, cb_scr, rgb_scr):
    @pl.when(pl.program_id(0) == 0)
    def _():
        for r in (hf_scr, cf_scr, cb_scr):
            r[...] = jnp.zeros((B, H), _f32)
        rgb_scr[...] = jnp.zeros((B, G4), _f32)

    rgb = rgb_scr[...]
    return hf_scr[...], cf_scr[...], cb_scr[...], rgb[:, :2 * H], rgb[:, 2 * H:]


def _scan0_kernel(gf_ref, gb_ref, wf_ref, wb_ref, hsf_ref, hsb_ref,
                  hf_scr, cf_scr, cb_scr, rgb_scr):
    carry0 = _init_or_load(hf_scr, cf_scr, cb_scr, rgb_scr)

    def store(s, h_f, h_b):
        hsf_ref[s] = h_f
        hsb_ref[CT - 1 - s] = h_b

    h_f, c_f, c_b, rgb0, rgb1 = _scan_steps(
        gf_ref, gb_ref, wf_ref[...], wb_ref[...], carry0, store)
    hf_scr[...], cf_scr[...], cb_scr[...] = h_f, c_f, c_b
    rgb_scr[...] = jnp.concatenate([rgb0, rgb1], axis=1)


def _scan1_kernel(hsf_f, hsb_f, hsf_r, hsb_r, wih_ref, bias_ref,
                  wf_ref, wb_ref, out_ref,
                  hf_scr, cf_scr, cb_scr, rgb_scr, hb_scr, gf_scr, gb_scr):
    wih = wih_ref[...]
    bias = bias_ref[...]

    def proj_strip(strip, cols, a_ref, b_ref, out_scr):
        rows = pl.ds(strip * UO, UO)
        a = a_ref[rows].reshape(UO * B, H).astype(_bf16)
        bb = b_ref[rows].reshape(UO * B, H).astype(_bf16)
        g = (jnp.dot(a, wih[:H, cols], preferred_element_type=_f32)
             + jnp.dot(bb, wih[H:, cols], preferred_element_type=_f32)
             + bias[:, cols])
        out_scr[rows] = g.reshape(UO, B, G4)

    fcols = slice(0, G4)
    bcols = slice(G4, 2 * G4)
    proj_strip(0, fcols, hsf_f, hsb_f, gf_scr)
    proj_strip(NS - 1, bcols, hsf_r, hsb_r, gb_scr)

    def extra(j):
        proj_strip(jnp.minimum(j + 1, NS - 1), fcols, hsf_f, hsb_f, gf_scr)
        proj_strip(jnp.maximum(NS - 2 - j, 0), bcols, hsf_r, hsb_r, gb_scr)

    carry0 = _init_or_load(hf_scr, cf_scr, cb_scr, rgb_scr)

    def store(s, h_f, h_b):
        hb_scr[...] = h_b

    h_f, c_f, c_b, rgb0, rgb1 = _scan_steps(
        gf_scr, gb_scr, wf_ref[...], wb_ref[...], carry0, store, extra=extra)
    hf_scr[...], cf_scr[...], cb_scr[...] = h_f, c_f, c_b
    rgb_scr[...] = jnp.concatenate([rgb0, rgb1], axis=1)
    out_ref[...] = jnp.concatenate([h_f, hb_scr[...]], axis=1)


@jax.jit
def _forward_impl(x, adj_matrix, params):
    w0 = jnp.concatenate(
        [params["W_ih_l0_d0"].T, params["W_ih_l0_d1"].T],
        axis=1).astype(_bf16)
    b0 = jnp.concatenate(
        [params["b_ih_l0_d0"] + params["b_hh_l0_d0"],
         params["b_ih_l0_d1"] + params["b_hh_l0_d1"]])[None, :]
    whh0_f = params["W_hh_l0_d0"].T.astype(_bf16)
    whh0_b = params["W_hh_l0_d1"].T.astype(_bf16)
    w1 = jnp.concatenate(
        [params["W_ih_l1_d0"].T, params["W_ih_l1_d1"].T],
        axis=1).astype(_bf16)
    b1 = jnp.concatenate(
        [params["b_ih_l1_d0"] + params["b_hh_l1_d0"],
         params["b_ih_l1_d1"] + params["b_hh_l1_d1"]])[None, :]
    whh1_f = params["W_hh_l1_d0"].T.astype(_bf16)
    whh1_b = params["W_hh_l1_d1"].T.astype(_bf16)

    x = x.astype(_f32)
    x_aug = jnp.concatenate(
        [x, jnp.ones((B, N, 1), _f32)], axis=2).astype(_bf16)

    gih0_f, gih0_b = pl.pallas_call(
        _agg_gih0_kernel,
        grid=(NT,),
        in_specs=[
            pl.BlockSpec((B, CT, N), lambda i: (0, i, 0)),
            pl.BlockSpec((B, N, D_IN + 1), lambda i: (0, 0, 0)),
            pl.BlockSpec((B, N, D_IN), lambda i: (0, 0, 0)),
            pl.BlockSpec((D_IN, 2 * G4), lambda i: (0, 0)),
            pl.BlockSpec((1, 2 * G4), lambda i: (0, 0)),
        ],
        out_specs=[
            pl.BlockSpec((CT, B, G4), lambda i: (i, 0, 0)),
            pl.BlockSpec((CT, B, G4), lambda i: (i, 0, 0)),
        ],
        out_shape=[
            jax.ShapeDtypeStruct((N, B, G4), _f32),
            jax.ShapeDtypeStruct((N, B, G4), _f32),
        ],
    )(adj_matrix, x_aug, x, w0, b0)

    rev = lambda i: (NT - 1 - i, 0, 0)
    fwd = lambda i: (i, 0, 0)
    hs_f, hs_b = pl.pallas_call(
        _scan0_kernel,
        grid=(NT,),
        in_specs=[
            pl.BlockSpec((CT, B, G4), fwd),
            pl.BlockSpec((CT, B, G4), rev),
            pl.BlockSpec((H, G4), lambda i: (0, 0)),
            pl.BlockSpec((H, G4), lambda i: (0, 0)),
        ],
        out_specs=[
            pl.BlockSpec((CT, B, H), fwd),
            pl.BlockSpec((CT, B, H), rev),
        ],
        out_shape=[
            jax.ShapeDtypeStruct((N, B, H), _f32),
            jax.ShapeDtypeStruct((N, B, H), _f32),
        ],
        scratch_shapes=[pltpu.VMEM((B, H), _f32) for _ in range(3)]
        + [pltpu.VMEM((B, G4), _f32)],
    )(gih0_f, gih0_b, whh0_f, whh0_b)

    out = pl.pallas_call(
        _scan1_kernel,
        grid=(NT,),
        in_specs=[
            pl.BlockSpec((CT, B, H), fwd),
            pl.BlockSpec((CT, B, H), fwd),
            pl.BlockSpec((CT, B, H), rev),
            pl.BlockSpec((CT, B, H), rev),
            pl.BlockSpec((2 * H, 2 * G4), lambda i: (0, 0)),
            pl.BlockSpec((1, 2 * G4), lambda i: (0, 0)),
            pl.BlockSpec((H, G4), lambda i: (0, 0)),
            pl.BlockSpec((H, G4), lambda i: (0, 0)),
        ],
        out_specs=pl.BlockSpec((B, 2 * H), lambda i: (0, 0)),
        out_shape=jax.ShapeDtypeStruct((B, 2 * H), _f32),
        scratch_shapes=(
            [pltpu.VMEM((B, H), _f32) for _ in range(3)]
            + [pltpu.VMEM((B, G4), _f32), pltpu.VMEM((B, H), _f32)]
            + [pltpu.VMEM((CT, B, G4), _f32), pltpu.VMEM((CT, B, G4), _f32)]
        ),
    )(hs_f, hs_b, hs_f, hs_b, w1, b1, whh1_f, whh1_b)

    return out


def kernel(x, adj_matrix, params):
    return _forward_impl(x, adj_matrix, params)


# speedup vs baseline: 1.0755x; 1.0755x over previous
import functools

import jax
import jax.numpy as jnp
from jax.experimental import pallas as pl
from jax.experimental.pallas import tpu as pltpu

B = 8
N = 1024
D_IN = 256
H = 128
G4 = 4 * H
CT = 256
NT = N // CT

_f32 = jnp.float32
_bf16 = jnp.bfloat16


def _agg_gih0_kernel(adj_ref, x_ref, w_ref, b_ref, gf_ref, gb_ref):
    i = pl.program_id(0)
    w = w_ref[...]
    for b in range(B):
        nz = adj_ref[b] > 0
        mask = nz.astype(_bf16)
        deg = jnp.sum(nz.astype(_f32), axis=1, keepdims=True)
        nsum = jnp.dot(mask, x_ref[b].astype(_bf16),
                       preferred_element_type=_f32)
        xb = x_ref[b, pl.ds(i * CT, CT), :]
        feat = (xb + nsum) / (1.0 + deg)
        g = jnp.dot(feat.astype(_bf16), w,
                    preferred_element_type=_f32) + b_ref[...]
        gf_ref[:, b, :] = g[:, :G4]
        gb_ref[:, b, :] = g[:, G4:]


def _sigmoid(x):
    return 0.5 + 0.5 * jnp.tanh(0.5 * x)


def _lstm_cell(g, c):
    i = _sigmoid(g[:, 0:H])
    f = _sigmoid(g[:, H:2 * H])
    gg = jnp.tanh(g[:, 2 * H:3 * H])
    o = _sigmoid(g[:, 3 * H:4 * H])
    c2 = f * c + i * gg
    h2 = o * jnp.tanh(c2)
    return h2, c2


def _scan_steps(gf_ref, gb_ref, wf, wb, carry0, store):
    def dot2(h, w):
        hb = h.astype(_bf16)
        return (jnp.dot(hb, w[:, :2 * H], preferred_element_type=_f32),
                jnp.dot(hb, w[:, 2 * H:], preferred_element_type=_f32))

    def cell2(rg0, rg1, g, c):
        i = _sigmoid(rg0[:, 0:H] + g[:, 0:H])
        f = _sigmoid(rg0[:, H:2 * H] + g[:, H:2 * H])
        gg = jnp.tanh(rg1[:, 0:H] + g[:, 2 * H:3 * H])
        o = _sigmoid(rg1[:, H:2 * H] + g[:, 3 * H:4 * H])
        c2 = f * c + i * gg
        return o * jnp.tanh(c2), c2

    def step(s, carry):
        h_f, c_f, c_b, rgb0, rgb1 = carry
        rgf0, rgf1 = dot2(h_f, wf)
        h_b, c_b = cell2(rgb0, rgb1, gb_ref[CT - 1 - s], c_b)
        rgb0, rgb1 = dot2(h_b, wb)
        h_f, c_f = cell2(rgf0, rgf1, gf_ref[s], c_f)
        if store is not None:
            store(s, h_f, h_b)
        return (h_f, c_f, c_b, rgb0, rgb1)

    return jax.lax.fori_loop(0, CT, step, carry0, unroll=16)


def _init_or_load(hf_scr, cf_scr, cb_scr, rgb_scr):
    @pl.when(pl.program_id(0) == 0)
    def _():
        for r in (hf_scr, cf_scr, cb_scr):
            r[...] = jnp.zeros((B, H), _f32)
        rgb_scr[...] = jnp.zeros((B, G4), _f32)

    rgb = rgb_scr[...]
    return hf_scr[...], cf_scr[...], cb_scr[...], rgb[:, :2 * H], rgb[:, 2 * H:]


def _scan0_kernel(gf_ref, gb_ref, wf_ref, wb_ref, hsf_ref, hsb_ref,
                  hf_scr, cf_scr, cb_scr, rgb_scr):
    carry0 = _init_or_load(hf_scr, cf_scr, cb_scr, rgb_scr)

    def store(s, h_f, h_b):
        hsf_ref[s] = h_f
        hsb_ref[CT - 1 - s] = h_b

    h_f, c_f, c_b, rgb0, rgb1 = _scan_steps(
        gf_ref, gb_ref, wf_ref[...], wb_ref[...], carry0, store)
    hf_scr[...], cf_scr[...], cb_scr[...] = h_f, c_f, c_b
    rgb_scr[...] = jnp.concatenate([rgb0, rgb1], axis=1)


def _scan1_kernel(hsf_f, hsb_f, hsf_r, hsb_r, wih_ref, bias_ref,
                  wf_ref, wb_ref, out_ref,
                  hf_scr, cf_scr, cb_scr, rgb_scr, hb_scr, gf_scr, gb_scr):
    wih = wih_ref[...]
    bias = bias_ref[...]

    def proj(a_ref, b_ref, cols):
        a = a_ref[...].reshape(CT * B, H).astype(_bf16)
        bb = b_ref[...].reshape(CT * B, H).astype(_bf16)
        g = (jnp.dot(a, wih[:H, cols], preferred_element_type=_f32)
             + jnp.dot(bb, wih[H:, cols], preferred_element_type=_f32)
             + bias[:, cols])
        return g.reshape(CT, B, G4)

    gf_scr[...] = proj(hsf_f, hsb_f, slice(0, G4))
    gb_scr[...] = proj(hsf_r, hsb_r, slice(G4, 2 * G4))

    carry0 = _init_or_load(hf_scr, cf_scr, cb_scr, rgb_scr)

    def store(s, h_f, h_b):
        hb_scr[...] = h_b

    h_f, c_f, c_b, rgb0, rgb1 = _scan_steps(
        gf_scr, gb_scr, wf_ref[...], wb_ref[...], carry0, store)
    hf_scr[...], cf_scr[...], cb_scr[...] = h_f, c_f, c_b
    rgb_scr[...] = jnp.concatenate([rgb0, rgb1], axis=1)
    out_ref[...] = jnp.concatenate([h_f, hb_scr[...]], axis=1)


@jax.jit
def _forward_impl(x, adj_matrix, params):
    w0 = jnp.concatenate(
        [params["W_ih_l0_d0"].T, params["W_ih_l0_d1"].T],
        axis=1).astype(_bf16)
    b0 = jnp.concatenate(
        [params["b_ih_l0_d0"] + params["b_hh_l0_d0"],
         params["b_ih_l0_d1"] + params["b_hh_l0_d1"]])[None, :]
    whh0_f = params["W_hh_l0_d0"].T.astype(_bf16)
    whh0_b = params["W_hh_l0_d1"].T.astype(_bf16)
    w1 = jnp.concatenate(
        [params["W_ih_l1_d0"].T, params["W_ih_l1_d1"].T],
        axis=1).astype(_bf16)
    b1 = jnp.concatenate(
        [params["b_ih_l1_d0"] + params["b_hh_l1_d0"],
         params["b_ih_l1_d1"] + params["b_hh_l1_d1"]])[None, :]
    whh1_f = params["W_hh_l1_d0"].T.astype(_bf16)
    whh1_b = params["W_hh_l1_d1"].T.astype(_bf16)

    x = x.astype(_f32)

    gih0_f, gih0_b = pl.pallas_call(
        _agg_gih0_kernel,
        grid=(NT,),
        in_specs=[
            pl.BlockSpec((B, CT, N), lambda i: (0, i, 0)),
            pl.BlockSpec((B, N, D_IN), lambda i: (0, 0, 0)),
            pl.BlockSpec((D_IN, 2 * G4), lambda i: (0, 0)),
            pl.BlockSpec((1, 2 * G4), lambda i: (0, 0)),
        ],
        out_specs=[
            pl.BlockSpec((CT, B, G4), lambda i: (i, 0, 0)),
            pl.BlockSpec((CT, B, G4), lambda i: (i, 0, 0)),
        ],
        out_shape=[
            jax.ShapeDtypeStruct((N, B, G4), _f32),
            jax.ShapeDtypeStruct((N, B, G4), _f32),
        ],
    )(adj_matrix, x, w0, b0)

    rev = lambda i: (NT - 1 - i, 0, 0)
    fwd = lambda i: (i, 0, 0)
    hs_f, hs_b = pl.pallas_call(
        _scan0_kernel,
        grid=(NT,),
        in_specs=[
            pl.BlockSpec((CT, B, G4), fwd),
            pl.BlockSpec((CT, B, G4), rev),
            pl.BlockSpec((H, G4), lambda i: (0, 0)),
            pl.BlockSpec((H, G4), lambda i: (0, 0)),
        ],
        out_specs=[
            pl.BlockSpec((CT, B, H), fwd),
            pl.BlockSpec((CT, B, H), rev),
        ],
        out_shape=[
            jax.ShapeDtypeStruct((N, B, H), _f32),
            jax.ShapeDtypeStruct((N, B, H), _f32),
        ],
        scratch_shapes=[pltpu.VMEM((B, H), _f32) for _ in range(3)]
        + [pltpu.VMEM((B, G4), _f32)],
    )(gih0_f, gih0_b, whh0_f, whh0_b)

    out = pl.pallas_call(
        _scan1_kernel,
        grid=(NT,),
        in_specs=[
            pl.BlockSpec((CT, B, H), fwd),
            pl.BlockSpec((CT, B, H), fwd),
            pl.BlockSpec((CT, B, H), rev),
            pl.BlockSpec((CT, B, H), rev),
            pl.BlockSpec((2 * H, 2 * G4), lambda i: (0, 0)),
            pl.BlockSpec((1, 2 * G4), lambda i: (0, 0)),
            pl.BlockSpec((H, G4), lambda i: (0, 0)),
            pl.BlockSpec((H, G4), lambda i: (0, 0)),
        ],
        out_specs=pl.BlockSpec((B, 2 * H), lambda i: (0, 0)),
        out_shape=jax.ShapeDtypeStruct((B, 2 * H), _f32),
        scratch_shapes=(
            [pltpu.VMEM((B, H), _f32) for _ in range(3)]
            + [pltpu.VMEM((B, G4), _f32), pltpu.VMEM((B, H), _f32)]
            + [pltpu.VMEM((CT, B, G4), _f32), pltpu.VMEM((CT, B, G4), _f32)]
        ),
    )(hs_f, hs_b, hs_f, hs_b, w1, b1, whh1_f, whh1_b)

    return out


def kernel(x, adj_matrix, params):
    return _forward_impl(x, adj_matrix, params)
